# baseline (speedup 1.0000x reference)
import jax
import jax.numpy as jnp
from jax import lax
from jax.experimental import pallas as pl
from jax.experimental.pallas import tpu as pltpu
from jax.experimental.pallas import tpu_sc as plsc

NU = 50000
NN = 100000
D = 32
DH = D // 2
E = 1600000
B = 16384

NC = 2
NS = 16
EC = 736
NCH = 138
EPT = EC * NCH
EPAD = EPT * NS
G16 = EC // 16


def _mesh():
    return plsc.VectorSubcoreMesh(
        core_axis_name="c", subcore_axis_name="s",
        num_cores=NC, num_subcores=NS)


def _layer_body(embA, embB, epr, outA, outB,
                rows0, rows1, ebuf0, ebuf1, ebuf2,
                acc, gsem, ssem, esem):
    c = lax.axis_index("c")
    s = lax.axis_index("s")
    rowsb = (rows0, rows1)
    ebufs = (ebuf0, ebuf1, ebuf2)

    zero = jnp.zeros((16,), jnp.float32)

    def zrow(i, _):
        rows0[i, pl.ds(0, 16)] = zero
        return 0

    lax.fori_loop(0, 400, zrow, 0)
    zbase = s * 6400

    def zcopy(k, _):
        pltpu.sync_copy(rows0.at[pl.ds(0, 400)],
                        acc.at[pl.ds(zbase + k * 400, 400)])
        return 0

    lax.fori_loop(0, 10, zcopy, 0)

    @pl.when(s < NS - 1)
    def _zero_tail():
        lax.fori_loop(10, 16, zcopy, 0)

    plsc.subcore_barrier()

    WSCALE = jnp.float32(2.0 ** -30)

    def fire_edges(i, eb):
        t = s * NCH + i
        pltpu.async_copy(epr.at[pl.ds(t * 3 * EC, 3 * EC)], eb, esem)

    def drain_edges(eb):
        pltpu.make_async_copy(epr.at[pl.ds(0, 3 * EC)], eb, esem).wait()

    def fire_gather(eb, rows):
        @pl.when(c == 0)
        def _ga():
            pltpu.async_copy(embA.at[eb.at[pl.ds(0, EC)]], rows, gsem)

        @pl.when(c == 1)
        def _gb():
            pltpu.async_copy(embB.at[eb.at[pl.ds(0, EC)]], rows, gsem)

    def drain_gather(rows):
        pltpu.make_async_copy(embA.at[pl.ds(0, EC)], rows, gsem).wait()

    def fire_scatter(eb, rows):
        pltpu.async_copy(rows, acc.at[eb.at[pl.ds(EC, EC)]], ssem, add=True)

    def drain_scatter(rows):
        pltpu.make_async_copy(embA.at[pl.ds(0, EC)], rows, ssem).wait()

    def emit(i, a, d):
        rowsA = rowsb[a]
        ebufA = ebufs[d]

        @pl.when(i < NCH - 1)
        def _prefetch():
            fire_edges(i + 1, ebufs[(d + 1) % 3])

        drain_gather(rowsA)

        @pl.when(i >= 1)
        def _drain_prev_scatter():
            drain_scatter(rowsb[1 - a])

        @pl.when(i < NCH - 1)
        def _next_gather():
            drain_edges(ebufs[(d + 1) % 3])
            fire_gather(ebufs[(d + 1) % 3], rowsb[1 - a])

        def scale(g, _):
            wv16 = (ebufA[pl.ds(2 * EC + g * 16, 16)].astype(jnp.float32)
                    * jnp.full((16,), WSCALE))
            for j in range(16):
                i_row = g * 16 + j
                rowsA[i_row, pl.ds(0, 16)] = (
                    rowsA[i_row, pl.ds(0, 16)] * jnp.full((16,), wv16[j]))
            return 0

        lax.fori_loop(0, G16, scale, 0)
        fire_scatter(ebufA, rowsA)

    fire_edges(0, ebufs[0])
    drain_edges(ebufs[0])
    fire_gather(ebufs[0], rows0)

    def six(p, _):
        for k in range(6):
            emit(6 * p + k, k % 2, k % 3)
        return 0

    lax.fori_loop(0, NCH // 6, six, 0)
    drain_scatter(rows1)

    plsc.subcore_barrier()

    def wb_to(out_hbm):
        def wbcopy(k, _):
            pltpu.sync_copy(acc.at[pl.ds(zbase + k * 400, 400)],
                            out_hbm.at[pl.ds(zbase + k * 400, 400)])
            return 0

        lax.fori_loop(0, 10, wbcopy, 0)

        @pl.when(s < NS - 1)
        def _wb_tail():
            lax.fori_loop(10, 16, wbcopy, 0)

    @pl.when(c == 0)
    def _wa():
        wb_to(outA)

    @pl.when(c == 1)
    def _wb():
        wb_to(outB)


def _final_body(e0a, e0b, e1a, e1b, e2a, e2b, e3a, e3b, usr, itm,
                uo2, io2, idxb, r0, r1, r2, r3, sem):
    c = lax.axis_index("c")
    s = lax.axis_index("s")
    rpt = B // NS

    def gather4(idx):
        @pl.when(c == 0)
        def _g0():
            pltpu.async_copy(e0a.at[idx], r0, sem)
            pltpu.async_copy(e1a.at[idx], r1, sem)
            pltpu.async_copy(e2a.at[idx], r2, sem)
            pltpu.async_copy(e3a.at[idx], r3, sem)

        @pl.when(c == 1)
        def _g1():
            pltpu.async_copy(e0b.at[idx], r0, sem)
            pltpu.async_copy(e1b.at[idx], r1, sem)
            pltpu.async_copy(e2b.at[idx], r2, sem)
            pltpu.async_copy(e3b.at[idx], r3, sem)

        for r in (r0, r1, r2, r3):
            pltpu.make_async_copy(e0a.at[pl.ds(0, rpt)], r, sem).wait()

    for ids_hbm, out2, off in ((usr, uo2, 0), (itm, io2, NU)):
        pltpu.sync_copy(ids_hbm.at[pl.ds(s * rpt, rpt)], idxb)
        if off:
            def addoff(g, _):
                idxb[pl.ds(g * 16, 16)] = idxb[pl.ds(g * 16, 16)] + off
                return 0
            lax.fori_loop(0, rpt // 16, addoff, 0)
        gather4(idxb)

        def avg(i, _):
            r0[i, pl.ds(0, 16)] = (
                r0[i, pl.ds(0, 16)] + r1[i, pl.ds(0, 16)]
                + r2[i, pl.ds(0, 16)] + r3[i, pl.ds(0, 16)]) * 0.25
            return 0

        lax.fori_loop(0, rpt, avg, 0)
        pltpu.sync_copy(r0, out2.at[pl.ds(c * B + s * rpt, rpt)])


def kernel(users, items, edge_index, edge_weight, user_emb, item_emb):
    pad = EPAD - E
    src = jnp.concatenate([edge_index[0], jnp.zeros((pad,), jnp.int32)])
    dst = jnp.concatenate([edge_index[1], jnp.zeros((pad,), jnp.int32)])
    wfix = jnp.concatenate(
        [jnp.round(edge_weight * jnp.float32(2.0 ** 30)).astype(jnp.int32),
         jnp.zeros((pad,), jnp.int32)])
    T = EPAD // EC
    ep = jnp.concatenate(
        [src.reshape(T, EC), dst.reshape(T, EC), wfix.reshape(T, EC)],
        axis=1).reshape(-1)
    e0a = jnp.concatenate([user_emb[:, :DH], item_emb[:, :DH]], axis=0)
    e0b = jnp.concatenate([user_emb[:, DH:], item_emb[:, DH:]], axis=0)

    cp = pltpu.CompilerParams(use_tc_tiling_on_sc=False)
    rowsbufs = [pltpu.VMEM((EC, DH), jnp.float32)] * 2
    ebufs = [pltpu.VMEM((3 * EC,), jnp.int32)] * 3
    layer = pl.kernel(
        _layer_body,
        out_type=(jax.ShapeDtypeStruct((NN, DH), jnp.float32),
                  jax.ShapeDtypeStruct((NN, DH), jnp.float32)),
        mesh=_mesh(),
        compiler_params=cp,
        scratch_types=[
            *rowsbufs, *ebufs,
            pltpu.VMEM_SHARED((NN, DH), jnp.float32),
            pltpu.SemaphoreType.DMA,
            pltpu.SemaphoreType.DMA,
            pltpu.SemaphoreType.DMA,
        ],
    )
    e1a, e1b = layer(e0a, e0b, ep)
    e2a, e2b = layer(e1a, e1b, ep)
    e3a, e3b = layer(e2a, e2b, ep)

    rpt = B // NS
    fin = pl.kernel(
        _final_body,
        out_type=(jax.ShapeDtypeStruct((2 * B, DH), jnp.float32),
                  jax.ShapeDtypeStruct((2 * B, DH), jnp.float32)),
        mesh=_mesh(),
        compiler_params=cp,
        scratch_types=[
            pltpu.VMEM((rpt,), jnp.int32),
            pltpu.VMEM((rpt, DH), jnp.float32),
            pltpu.VMEM((rpt, DH), jnp.float32),
            pltpu.VMEM((rpt, DH), jnp.float32),
            pltpu.VMEM((rpt, DH), jnp.float32),
            pltpu.SemaphoreType.DMA,
        ],
    )
    uo2, io2 = fin(e0a, e0b, e1a, e1b, e2a, e2b, e3a, e3b, users, items)
    uo = jnp.concatenate([uo2[:B], uo2[B:]], axis=1)
    io = jnp.concatenate([io2[:B], io2[B:]], axis=1)
    return uo, io


# speedup vs baseline: 1.0544x; 1.0544x over previous
import jax
import jax.numpy as jnp
from jax import lax
from jax.experimental import pallas as pl
from jax.experimental.pallas import tpu as pltpu
from jax.experimental.pallas import tpu_sc as plsc

NU = 50000
NN = 100000
D = 32
DH = D // 2
E = 1600000
B = 16384

NC = 2
NS = 16
EC = 768
NCH = 132
EPT = EC * NCH
EPAD = EPT * NS
G16 = EC // 16


def _mesh():
    return plsc.VectorSubcoreMesh(
        core_axis_name="c", subcore_axis_name="s",
        num_cores=NC, num_subcores=NS)


def _layer_body(embA, embB, srcr, dstr, wr, outA, outB,
                srcb0, wb0, rows0, srcb1, wb1, rows1,
                dstb0, dstb1, dstb2,
                acc, gsem, ssem, esem):
    c = lax.axis_index("c")
    s = lax.axis_index("s")
    bufs = ((srcb0, wb0, rows0), (srcb1, wb1, rows1))
    dstbs = (dstb0, dstb1, dstb2)

    zero = jnp.zeros((16,), jnp.float32)

    def zrow(i, _):
        rows0[i, pl.ds(0, 16)] = zero
        return 0

    lax.fori_loop(0, 400, zrow, 0)
    zbase = s * 6400

    def zcopy(k, _):
        pltpu.sync_copy(rows0.at[pl.ds(0, 400)],
                        acc.at[pl.ds(zbase + k * 400, 400)])
        return 0

    lax.fori_loop(0, 10, zcopy, 0)

    @pl.when(s < NS - 1)
    def _zero_tail():
        lax.fori_loop(10, 16, zcopy, 0)

    plsc.subcore_barrier()

    def fire_edges(i, bset, dstb):
        srcb, wb, _ = bset
        t = s * NCH + i
        pltpu.async_copy(srcr.at[pl.ds(t * EC, EC)], srcb, esem)
        pltpu.async_copy(dstr.at[pl.ds(t * EC, EC)], dstb, esem)
        pltpu.async_copy(wr.at[pl.ds(t * EC, EC)], wb, esem)

    def drain_edges(bset, dstb):
        srcb, wb, _ = bset
        pltpu.make_async_copy(srcr.at[pl.ds(0, EC)], srcb, esem).wait()
        pltpu.make_async_copy(dstr.at[pl.ds(0, EC)], dstb, esem).wait()
        pltpu.make_async_copy(wr.at[pl.ds(0, EC)], wb, esem).wait()

    H = EC // 2

    def fire_gather(bset):
        srcb, _, rows = bset

        @pl.when(c == 0)
        def _ga():
            pltpu.async_copy(embA.at[srcb.at[pl.ds(0, H)]],
                             rows.at[pl.ds(0, H)], gsem)
            pltpu.async_copy(embA.at[srcb.at[pl.ds(H, H)]],
                             rows.at[pl.ds(H, H)], gsem)

        @pl.when(c == 1)
        def _gb():
            pltpu.async_copy(embB.at[srcb.at[pl.ds(0, H)]],
                             rows.at[pl.ds(0, H)], gsem)
            pltpu.async_copy(embB.at[srcb.at[pl.ds(H, H)]],
                             rows.at[pl.ds(H, H)], gsem)

    def drain_gather(bset):
        pltpu.make_async_copy(embA.at[pl.ds(0, EC)], bset[2], gsem).wait()

    def fire_scatter(bset, dstb):
        rows = bset[2]
        pltpu.async_copy(rows, acc.at[dstb], ssem, add=True)

    def drain_scatter(bset):
        pltpu.make_async_copy(embA.at[pl.ds(0, EC)], bset[2], ssem).wait()

    def emit(i, a, d):
        A = bufs[a]
        B_ = bufs[1 - a]
        _, wbA, rowsA = A

        @pl.when(i < NCH - 1)
        def _prefetch():
            fire_edges(i + 1, B_, dstbs[(d + 1) % 3])

        drain_gather(A)

        @pl.when(i >= 1)
        def _drain_prev_scatter():
            drain_scatter(B_)

        @pl.when(i < NCH - 1)
        def _next_gather():
            drain_edges(B_, dstbs[(d + 1) % 3])
            fire_gather(B_)

        def scale(g, _):
            wv16 = wbA[pl.ds(g * 16, 16)]
            for j in range(16):
                i_row = g * 16 + j
                rowsA[i_row, pl.ds(0, 16)] = (
                    rowsA[i_row, pl.ds(0, 16)] * jnp.full((16,), wv16[j]))
            return 0

        lax.fori_loop(0, G16, scale, 0)
        fire_scatter(A, dstbs[d])

    fire_edges(0, bufs[0], dstbs[0])
    drain_edges(bufs[0], dstbs[0])
    fire_gather(bufs[0])

    def six(p, _):
        for k in range(6):
            emit(6 * p + k, k % 2, k % 3)
        return 0

    lax.fori_loop(0, NCH // 6, six, 0)
    drain_scatter(bufs[1])

    plsc.subcore_barrier()

    def wb_to(out_hbm):
        def wbcopy(k, _):
            pltpu.sync_copy(acc.at[pl.ds(zbase + k * 400, 400)],
                            out_hbm.at[pl.ds(zbase + k * 400, 400)])
            return 0

        lax.fori_loop(0, 10, wbcopy, 0)

        @pl.when(s < NS - 1)
        def _wb_tail():
            lax.fori_loop(10, 16, wbcopy, 0)

    @pl.when(c == 0)
    def _wa():
        wb_to(outA)

    @pl.when(c == 1)
    def _wb():
        wb_to(outB)


def _final_body(e0a, e0b, e1a, e1b, e2a, e2b, e3a, e3b, usr, itm,
                uo2, io2, idxb, r0, r1, r2, r3, sem):
    c = lax.axis_index("c")
    s = lax.axis_index("s")
    rpt = B // NS

    def gather4(idx):
        @pl.when(c == 0)
        def _g0():
            pltpu.async_copy(e0a.at[idx], r0, sem)
            pltpu.async_copy(e1a.at[idx], r1, sem)
            pltpu.async_copy(e2a.at[idx], r2, sem)
            pltpu.async_copy(e3a.at[idx], r3, sem)

        @pl.when(c == 1)
        def _g1():
            pltpu.async_copy(e0b.at[idx], r0, sem)
            pltpu.async_copy(e1b.at[idx], r1, sem)
            pltpu.async_copy(e2b.at[idx], r2, sem)
            pltpu.async_copy(e3b.at[idx], r3, sem)

        for r in (r0, r1, r2, r3):
            pltpu.make_async_copy(e0a.at[pl.ds(0, rpt)], r, sem).wait()

    for ids_hbm, out2, off in ((usr, uo2, 0), (itm, io2, NU)):
        pltpu.sync_copy(ids_hbm.at[pl.ds(s * rpt, rpt)], idxb)
        if off:
            def addoff(g, _):
                idxb[pl.ds(g * 16, 16)] = idxb[pl.ds(g * 16, 16)] + off
                return 0
            lax.fori_loop(0, rpt // 16, addoff, 0)
        gather4(idxb)

        def avg(i, _):
            r0[i, pl.ds(0, 16)] = (
                r0[i, pl.ds(0, 16)] + r1[i, pl.ds(0, 16)]
                + r2[i, pl.ds(0, 16)] + r3[i, pl.ds(0, 16)]) * 0.25
            return 0

        lax.fori_loop(0, rpt, avg, 0)
        pltpu.sync_copy(r0, out2.at[pl.ds(c * B + s * rpt, rpt)])


def kernel(users, items, edge_index, edge_weight, user_emb, item_emb):
    pad = EPAD - E
    src = jnp.concatenate([edge_index[0], jnp.zeros((pad,), jnp.int32)])
    dst = jnp.concatenate([edge_index[1], jnp.zeros((pad,), jnp.int32)])
    w = jnp.concatenate([edge_weight, jnp.zeros((pad,), jnp.float32)])
    e0a = jnp.concatenate([user_emb[:, :DH], item_emb[:, :DH]], axis=0)
    e0b = jnp.concatenate([user_emb[:, DH:], item_emb[:, DH:]], axis=0)

    cp = pltpu.CompilerParams(use_tc_tiling_on_sc=False)
    bufset = [
        pltpu.VMEM((EC,), jnp.int32),
        pltpu.VMEM((EC,), jnp.float32),
        pltpu.VMEM((EC, DH), jnp.float32),
    ]
    dstbufs = [pltpu.VMEM((EC,), jnp.int32)] * 3
    layer = pl.kernel(
        _layer_body,
        out_type=(jax.ShapeDtypeStruct((NN, DH), jnp.float32),
                  jax.ShapeDtypeStruct((NN, DH), jnp.float32)),
        mesh=_mesh(),
        compiler_params=cp,
        scratch_types=[
            *bufset, *bufset, *dstbufs,
            pltpu.VMEM_SHARED((NN, DH), jnp.float32),
            pltpu.SemaphoreType.DMA,
            pltpu.SemaphoreType.DMA,
            pltpu.SemaphoreType.DMA,
        ],
    )
    e1a, e1b = layer(e0a, e0b, src, dst, w)
    e2a, e2b = layer(e1a, e1b, src, dst, w)
    e3a, e3b = layer(e2a, e2b, src, dst, w)

    rpt = B // NS
    fin = pl.kernel(
        _final_body,
        out_type=(jax.ShapeDtypeStruct((2 * B, DH), jnp.float32),
                  jax.ShapeDtypeStruct((2 * B, DH), jnp.float32)),
        mesh=_mesh(),
        compiler_params=cp,
        scratch_types=[
            pltpu.VMEM((rpt,), jnp.int32),
            pltpu.VMEM((rpt, DH), jnp.float32),
            pltpu.VMEM((rpt, DH), jnp.float32),
            pltpu.VMEM((rpt, DH), jnp.float32),
            pltpu.VMEM((rpt, DH), jnp.float32),
            pltpu.SemaphoreType.DMA,
        ],
    )
    uo2, io2 = fin(e0a, e0b, e1a, e1b, e2a, e2b, e3a, e3b, users, items)
    uo = jnp.concatenate([uo2[:B], uo2[B:]], axis=1)
    io = jnp.concatenate([io2[:B], io2[B:]], axis=1)
    return uo, io
